# baseline (speedup 1.0000x reference)
import functools

import jax
import jax.numpy as jnp
from jax import lax
from jax.experimental import pallas as pl
from jax.experimental.pallas import tpu as pltpu
from jax.experimental.pallas import tpu_sc as plsc

_LANES = 16


def _transpose_table(emb):
    V, H = emb.shape
    VC = 512
    grid = (V + VC - 1) // VC

    def body(src_ref, dst_ref):
        y = src_ref[...].T.reshape(VC // 4, 4, H)
        for q in range(4):
            dst_ref[:, q * H:(q + 1) * H] = y[:, q, :]

    out = pl.pallas_call(
        body,
        grid=(grid,),
        in_specs=[pl.BlockSpec((H, VC), lambda i: (0, i))],
        out_specs=pl.BlockSpec((VC // 4, 4 * H), lambda i: (i, 0)),
        out_shape=jax.ShapeDtypeStruct((V // 4, 4 * H), jnp.float32),
    )(emb.T)
    return out.reshape(V, H)


def _make_kernel(B, L, V, H, NW):
    assert B % NW == 0 and H == 32
    BB = B // NW
    HT = H // 8
    mesh = plsc.VectorSubcoreMesh(core_axis_name="c", subcore_axis_name="s")
    NC = mesh.num_cores

    @functools.partial(
        pl.kernel,
        out_type=jax.ShapeDtypeStruct((L, HT, NW, 8, BB), jnp.float32),
        mesh=mesh,
        scratch_types=[
            pltpu.VMEM((L, BB), jnp.int32),
            pltpu.VMEM((L, H), jnp.float32),
            pltpu.VMEM((BB, H), jnp.float32),
            pltpu.VMEM((BB, H), jnp.float32),
            pltpu.VMEM((H, BB + 9), jnp.float32),
            pltpu.VMEM((H, BB + 9), jnp.float32),
            pltpu.SemaphoreType.DMA,
            pltpu.SemaphoreType.DMA,
            pltpu.SemaphoreType.DMA,
            pltpu.SemaphoreType.DMA,
        ],
        compiler_params=pltpu.CompilerParams(
            use_tc_tiling_on_sc=False, needs_layout_passes=False),
    )
    def k(x_hbm, emb_hbm, pos_hbm, out_hbm, idx_all, pos_v,
          rb0, rb1, tb0, tb1, sg0, sg1, sw0, sw1):
        cid = lax.axis_index("c")
        sid = lax.axis_index("s")
        wid = sid * NC + cid

        pltpu.sync_copy(pos_hbm, pos_v)
        pltpu.sync_copy(x_hbm.at[wid], idx_all)

        hrow0 = lax.iota(jnp.int32, _LANES)
        hrow1 = hrow0 + _LANES
        slots = ((rb0, tb0, sg0, sw0), (rb1, tb1, sg1, sw1))

        def issue_gather(l, rb, sg):
            pltpu.async_copy(emb_hbm.at[idx_all.at[l]], rb, sg)

        def wait_gather(rb, sg):
            pltpu.make_async_copy(emb_hbm.at[pl.ds(0, BB)], rb, sg).wait()

        def transpose_add(l, rb, tb):
            p0 = pos_v[l, pl.ds(0, _LANES)]
            p1 = pos_v[l, pl.ds(_LANES, _LANES)]

            @pl.loop(0, BB, unroll=16)
            def _tok(b):
                bcol = jnp.full((_LANES,), 0, jnp.int32) + b
                plsc.store_scatter(tb, [hrow0, bcol],
                                   rb[b, pl.ds(0, _LANES)] + p0)
                plsc.store_scatter(tb, [hrow1, bcol],
                                   rb[b, pl.ds(_LANES, _LANES)] + p1)

        def issue_write(l, tb, sw):
            for ht in range(HT):
                pltpu.async_copy(tb.at[pl.ds(8 * ht, 8), pl.ds(0, BB)],
                                 out_hbm.at[l, ht, wid], sw)

        def wait_write(tb, sw):
            for ht in range(HT):
                pltpu.make_async_copy(tb.at[pl.ds(8 * ht, 8), pl.ds(0, BB)],
                                      out_hbm.at[0, ht, wid], sw).wait()

        issue_gather(0, rb0, sg0)
        issue_gather(1, rb1, sg1)
        for s in (0, 1):
            rb, tb, sg, sw = slots[s]
            wait_gather(rb, sg)
            transpose_add(s, rb, tb)
            issue_write(s, tb, sw)
            issue_gather(s + 2, rb, sg)

        @pl.loop(2, L - 2, step=2)
        def _pair(g):
            for s in (0, 1):
                rb, tb, sg, sw = slots[s]
                l = g + s
                wait_gather(rb, sg)
                wait_write(tb, sw)
                transpose_add(l, rb, tb)
                issue_write(l, tb, sw)
                issue_gather(l + 2, rb, sg)

        for s in (0, 1):
            rb, tb, sg, sw = slots[s]
            l = L - 2 + s
            wait_gather(rb, sg)
            wait_write(tb, sw)
            transpose_add(l, rb, tb)
            issue_write(l, tb, sw)
        for s in (0, 1):
            rb, tb, sg, sw = slots[s]
            wait_write(tb, sw)

    return k


def kernel(x, emb, pos_emb):
    B, L = x.shape
    V, H = emb.shape
    NW = 32
    BB = B // NW
    x_bt = x.astype(jnp.int32).T.reshape(L, NW, BB).transpose(1, 0, 2)
    emb_lin = _transpose_table(emb)
    out5 = _make_kernel(B, L, V, H, NW)(x_bt, emb_lin, pos_emb)
    return out5.transpose(2, 4, 0, 1, 3).reshape(B, L, H)


# speedup vs baseline: 1.0372x; 1.0372x over previous
import functools

import jax
import jax.numpy as jnp
from jax import lax
from jax.experimental import pallas as pl
from jax.experimental.pallas import tpu as pltpu
from jax.experimental.pallas import tpu_sc as plsc

_LANES = 16


def _transpose_table(emb):
    V, H = emb.shape
    VC = 512
    grid = (V + VC - 1) // VC
    rows = grid * (VC // 4)

    def body(src_ref, dst_ref):
        y = src_ref[...].T
        for q in range(4):
            dst_ref[:, q * H:(q + 1) * H] = y[q * (VC // 4):(q + 1) * (VC // 4), :]

    out = pl.pallas_call(
        body,
        grid=(grid,),
        in_specs=[pl.BlockSpec((H, VC), lambda i: (0, i))],
        out_specs=pl.BlockSpec((VC // 4, 4 * H), lambda i: (i, 0)),
        out_shape=jax.ShapeDtypeStruct((rows, 4 * H), jnp.float32),
    )(emb.T)
    return out.reshape(rows * 4, H)


def _make_kernel(B, L, V, H, NW):
    assert B % NW == 0 and H == 32
    BB = B // NW
    HT = H // 8
    mesh = plsc.VectorSubcoreMesh(core_axis_name="c", subcore_axis_name="s")
    NC = mesh.num_cores

    @functools.partial(
        pl.kernel,
        out_type=jax.ShapeDtypeStruct((L, HT, NW, 8, BB), jnp.float32),
        mesh=mesh,
        scratch_types=[
            pltpu.VMEM((L, BB), jnp.int32),
            pltpu.VMEM((L, H), jnp.float32),
            pltpu.VMEM((BB, H), jnp.float32),
            pltpu.VMEM((BB, H), jnp.float32),
            pltpu.VMEM((H, BB + 9), jnp.float32),
            pltpu.VMEM((H, BB + 9), jnp.float32),
            pltpu.SemaphoreType.DMA,
            pltpu.SemaphoreType.DMA,
            pltpu.SemaphoreType.DMA,
            pltpu.SemaphoreType.DMA,
        ],
        compiler_params=pltpu.CompilerParams(
            use_tc_tiling_on_sc=False, needs_layout_passes=False),
    )
    def k(x_hbm, emb_hbm, pos_hbm, out_hbm, idx_all, pos_v,
          rb0, rb1, tb0, tb1, sg0, sg1, sw0, sw1):
        cid = lax.axis_index("c")
        sid = lax.axis_index("s")
        wid = sid * NC + cid

        pltpu.sync_copy(pos_hbm, pos_v)
        pltpu.sync_copy(x_hbm.at[wid], idx_all)

        hrow0 = lax.iota(jnp.int32, _LANES)
        hrow1 = hrow0 + _LANES
        slots = ((rb0, tb0, sg0, sw0), (rb1, tb1, sg1, sw1))

        def issue_gather(l, rb, sg):
            pltpu.async_copy(emb_hbm.at[idx_all.at[l]], rb, sg)

        def wait_gather(rb, sg):
            pltpu.make_async_copy(emb_hbm.at[pl.ds(0, BB)], rb, sg).wait()

        def transpose_add(l, rb, tb):
            p0 = pos_v[l, pl.ds(0, _LANES)]
            p1 = pos_v[l, pl.ds(_LANES, _LANES)]

            @pl.loop(0, BB, unroll=16)
            def _tok(b):
                bcol = jnp.full((_LANES,), 0, jnp.int32) + b
                plsc.store_scatter(tb, [hrow0, bcol],
                                   rb[b, pl.ds(0, _LANES)] + p0)
                plsc.store_scatter(tb, [hrow1, bcol],
                                   rb[b, pl.ds(_LANES, _LANES)] + p1)

        def issue_write(l, tb, sw):
            for ht in range(HT):
                pltpu.async_copy(tb.at[pl.ds(8 * ht, 8), pl.ds(0, BB)],
                                 out_hbm.at[l, ht, wid], sw)

        def wait_write(tb, sw):
            for ht in range(HT):
                pltpu.make_async_copy(tb.at[pl.ds(8 * ht, 8), pl.ds(0, BB)],
                                      out_hbm.at[0, ht, wid], sw).wait()

        issue_gather(0, rb0, sg0)
        issue_gather(1, rb1, sg1)
        for s in (0, 1):
            rb, tb, sg, sw = slots[s]
            wait_gather(rb, sg)
            transpose_add(s, rb, tb)
            issue_write(s, tb, sw)
            issue_gather(s + 2, rb, sg)

        @pl.loop(2, L - 2, step=2)
        def _pair(g):
            for s in (0, 1):
                rb, tb, sg, sw = slots[s]
                l = g + s
                wait_gather(rb, sg)
                wait_write(tb, sw)
                transpose_add(l, rb, tb)
                issue_write(l, tb, sw)
                issue_gather(l + 2, rb, sg)

        for s in (0, 1):
            rb, tb, sg, sw = slots[s]
            l = L - 2 + s
            wait_gather(rb, sg)
            wait_write(tb, sw)
            transpose_add(l, rb, tb)
            issue_write(l, tb, sw)
        for s in (0, 1):
            rb, tb, sg, sw = slots[s]
            wait_write(tb, sw)

    return k


def kernel(x, emb, pos_emb):
    B, L = x.shape
    V, H = emb.shape
    NW = 32
    BB = B // NW
    xi = x.astype(jnp.int32)
    xi = (xi & -512) | ((xi & 127) << 2) | ((xi >> 7) & 3)
    x_bt = xi.T.reshape(L, NW, BB).transpose(1, 0, 2)
    emb_lin = _transpose_table(emb)
    out5 = _make_kernel(B, L, V, H, NW)(x_bt, emb_lin, pos_emb)
    return out5.transpose(2, 4, 0, 1, 3).reshape(B, L, H)


# speedup vs baseline: 2.6868x; 2.5903x over previous
import functools

import jax
import jax.numpy as jnp
from jax import lax
from jax.experimental import pallas as pl
from jax.experimental.pallas import tpu as pltpu
from jax.experimental.pallas import tpu_sc as plsc

_LANES = 16


def _transpose_table(emb):
    V, H = emb.shape
    VC = 8192
    SUB = 512
    grid = (V + VC - 1) // VC
    rows = grid * (VC // 4)

    def body(src_ref, dst_ref):
        for c in range(VC // SUB):
            y = src_ref[:, SUB * c:SUB * (c + 1)].T
            for q in range(4):
                dst_ref[pl.ds(128 * c, 128), q * H:(q + 1) * H] = (
                    y[q * 128:(q + 1) * 128, :])

    out = pl.pallas_call(
        body,
        grid=(grid,),
        in_specs=[pl.BlockSpec((H, VC), lambda i: (0, i))],
        out_specs=pl.BlockSpec((VC // 4, 4 * H), lambda i: (i, 0)),
        out_shape=jax.ShapeDtypeStruct((rows, 4 * H), jnp.float32),
    )(emb.T)
    return out.reshape(rows * 4, H)


def _make_kernel(B, L, V, H, NW):
    assert B % NW == 0 and H == 32
    BB = B // NW
    HT = H // 8
    mesh = plsc.VectorSubcoreMesh(core_axis_name="c", subcore_axis_name="s")
    NC = mesh.num_cores

    @functools.partial(
        pl.kernel,
        out_type=jax.ShapeDtypeStruct((L, HT, NW, 8, BB), jnp.float32),
        mesh=mesh,
        scratch_types=[
            pltpu.VMEM((L, BB), jnp.int32),
            pltpu.VMEM((L, H), jnp.float32),
            pltpu.VMEM((BB, H), jnp.float32),
            pltpu.VMEM((BB, H), jnp.float32),
            pltpu.VMEM((H, BB + 9), jnp.float32),
            pltpu.VMEM((H, BB + 9), jnp.float32),
            pltpu.SemaphoreType.DMA,
            pltpu.SemaphoreType.DMA,
            pltpu.SemaphoreType.DMA,
            pltpu.SemaphoreType.DMA,
        ],
        compiler_params=pltpu.CompilerParams(
            use_tc_tiling_on_sc=False, needs_layout_passes=False),
    )
    def k(x_hbm, emb_hbm, pos_hbm, out_hbm, idx_all, pos_v,
          rb0, rb1, tb0, tb1, sg0, sg1, sw0, sw1):
        cid = lax.axis_index("c")
        sid = lax.axis_index("s")
        wid = sid * NC + cid

        pltpu.sync_copy(pos_hbm, pos_v)
        pltpu.sync_copy(x_hbm.at[wid], idx_all)

        hrow0 = lax.iota(jnp.int32, _LANES)
        hrow1 = hrow0 + _LANES
        slots = ((rb0, tb0, sg0, sw0), (rb1, tb1, sg1, sw1))

        def issue_gather(l, rb, sg):
            pltpu.async_copy(emb_hbm.at[idx_all.at[l]], rb, sg)

        def wait_gather(rb, sg):
            pltpu.make_async_copy(emb_hbm.at[pl.ds(0, BB)], rb, sg).wait()

        def transpose_add(l, rb, tb):
            p0 = pos_v[l, pl.ds(0, _LANES)]
            p1 = pos_v[l, pl.ds(_LANES, _LANES)]

            @pl.loop(0, BB, unroll=16)
            def _tok(b):
                bcol = jnp.full((_LANES,), 0, jnp.int32) + b
                plsc.store_scatter(tb, [hrow0, bcol],
                                   rb[b, pl.ds(0, _LANES)] + p0)
                plsc.store_scatter(tb, [hrow1, bcol],
                                   rb[b, pl.ds(_LANES, _LANES)] + p1)

        def issue_write(l, tb, sw):
            for ht in range(HT):
                pltpu.async_copy(tb.at[pl.ds(8 * ht, 8), pl.ds(0, BB)],
                                 out_hbm.at[l, ht, wid], sw)

        def wait_write(tb, sw):
            for ht in range(HT):
                pltpu.make_async_copy(tb.at[pl.ds(8 * ht, 8), pl.ds(0, BB)],
                                      out_hbm.at[0, ht, wid], sw).wait()

        issue_gather(0, rb0, sg0)
        issue_gather(1, rb1, sg1)
        for s in (0, 1):
            rb, tb, sg, sw = slots[s]
            wait_gather(rb, sg)
            transpose_add(s, rb, tb)
            issue_write(s, tb, sw)
            issue_gather(s + 2, rb, sg)

        @pl.loop(2, L - 2, step=2)
        def _pair(g):
            for s in (0, 1):
                rb, tb, sg, sw = slots[s]
                l = g + s
                wait_gather(rb, sg)
                wait_write(tb, sw)
                transpose_add(l, rb, tb)
                issue_write(l, tb, sw)
                issue_gather(l + 2, rb, sg)

        for s in (0, 1):
            rb, tb, sg, sw = slots[s]
            l = L - 2 + s
            wait_gather(rb, sg)
            wait_write(tb, sw)
            transpose_add(l, rb, tb)
            issue_write(l, tb, sw)
        for s in (0, 1):
            rb, tb, sg, sw = slots[s]
            wait_write(tb, sw)

    return k


def kernel(x, emb, pos_emb):
    B, L = x.shape
    V, H = emb.shape
    NW = 32
    BB = B // NW
    xi = x.astype(jnp.int32)
    xi = (xi & -512) | ((xi & 127) << 2) | ((xi >> 7) & 3)
    x_bt = xi.T.reshape(L, NW, BB).transpose(1, 0, 2)
    emb_lin = _transpose_table(emb)
    out5 = _make_kernel(B, L, V, H, NW)(x_bt, emb_lin, pos_emb)
    return out5.transpose(2, 4, 0, 1, 3).reshape(B, L, H)


# speedup vs baseline: 2.7062x; 1.0072x over previous
import functools

import jax
import jax.numpy as jnp
from jax import lax
from jax.experimental import pallas as pl
from jax.experimental.pallas import tpu as pltpu
from jax.experimental.pallas import tpu_sc as plsc

_LANES = 16


def _transpose_table(emb):
    V, H = emb.shape
    VC = 16384
    SUB = 512
    grid = (V + VC - 1) // VC
    rows = grid * (VC // 4)

    def body(src_ref, dst_ref):
        for c in range(VC // SUB):
            y = src_ref[:, SUB * c:SUB * (c + 1)].T
            for q in range(4):
                dst_ref[pl.ds(128 * c, 128), q * H:(q + 1) * H] = (
                    y[q * 128:(q + 1) * 128, :])

    out = pl.pallas_call(
        body,
        grid=(grid,),
        in_specs=[pl.BlockSpec((H, VC), lambda i: (0, i))],
        out_specs=pl.BlockSpec((VC // 4, 4 * H), lambda i: (i, 0)),
        out_shape=jax.ShapeDtypeStruct((rows, 4 * H), jnp.float32),
    )(emb.T)
    return out.reshape(rows * 4, H)


def _make_kernel(B, L, V, H, NW):
    assert B % NW == 0 and H == 32
    BB = B // NW
    HT = H // 8
    mesh = plsc.VectorSubcoreMesh(core_axis_name="c", subcore_axis_name="s")
    NC = mesh.num_cores

    @functools.partial(
        pl.kernel,
        out_type=jax.ShapeDtypeStruct((L, HT, NW, 8, BB), jnp.float32),
        mesh=mesh,
        scratch_types=[
            pltpu.VMEM((L, BB), jnp.int32),
            pltpu.VMEM((L, H), jnp.float32),
            pltpu.VMEM((BB, H), jnp.float32),
            pltpu.VMEM((BB, H), jnp.float32),
            pltpu.VMEM((H, BB + 9), jnp.float32),
            pltpu.VMEM((H, BB + 9), jnp.float32),
            pltpu.SemaphoreType.DMA,
            pltpu.SemaphoreType.DMA,
            pltpu.SemaphoreType.DMA,
            pltpu.SemaphoreType.DMA,
        ],
        compiler_params=pltpu.CompilerParams(
            use_tc_tiling_on_sc=False, needs_layout_passes=False),
    )
    def k(x_hbm, emb_hbm, pos_hbm, out_hbm, idx_all, pos_v,
          rb0, rb1, tb0, tb1, sg0, sg1, sw0, sw1):
        cid = lax.axis_index("c")
        sid = lax.axis_index("s")
        wid = sid * NC + cid

        pltpu.sync_copy(pos_hbm, pos_v)
        pltpu.sync_copy(x_hbm.at[wid], idx_all)

        hrow0 = lax.iota(jnp.int32, _LANES)
        hrow1 = hrow0 + _LANES
        slots = ((rb0, tb0, sg0, sw0), (rb1, tb1, sg1, sw1))

        def issue_gather(l, rb, sg):
            pltpu.async_copy(emb_hbm.at[idx_all.at[l]], rb, sg)

        def wait_gather(rb, sg):
            pltpu.make_async_copy(emb_hbm.at[pl.ds(0, BB)], rb, sg).wait()

        def transpose_add(l, rb, tb):
            p0 = pos_v[l, pl.ds(0, _LANES)]
            p1 = pos_v[l, pl.ds(_LANES, _LANES)]

            @pl.loop(0, BB, unroll=16)
            def _tok(b):
                bcol = jnp.full((_LANES,), 0, jnp.int32) + b
                plsc.store_scatter(tb, [hrow0, bcol],
                                   rb[b, pl.ds(0, _LANES)] + p0)
                plsc.store_scatter(tb, [hrow1, bcol],
                                   rb[b, pl.ds(_LANES, _LANES)] + p1)

        def issue_write(l, tb, sw):
            for ht in range(HT):
                pltpu.async_copy(tb.at[pl.ds(8 * ht, 8), pl.ds(0, BB)],
                                 out_hbm.at[l, ht, wid], sw)

        def wait_write(tb, sw):
            for ht in range(HT):
                pltpu.make_async_copy(tb.at[pl.ds(8 * ht, 8), pl.ds(0, BB)],
                                      out_hbm.at[0, ht, wid], sw).wait()

        issue_gather(0, rb0, sg0)
        issue_gather(1, rb1, sg1)
        for s in (0, 1):
            rb, tb, sg, sw = slots[s]
            wait_gather(rb, sg)
            transpose_add(s, rb, tb)
            issue_write(s, tb, sw)
            issue_gather(s + 2, rb, sg)

        @pl.loop(2, L - 2, step=2)
        def _pair(g):
            for s in (0, 1):
                rb, tb, sg, sw = slots[s]
                l = g + s
                wait_gather(rb, sg)
                wait_write(tb, sw)
                transpose_add(l, rb, tb)
                issue_write(l, tb, sw)
                issue_gather(l + 2, rb, sg)

        for s in (0, 1):
            rb, tb, sg, sw = slots[s]
            l = L - 2 + s
            wait_gather(rb, sg)
            wait_write(tb, sw)
            transpose_add(l, rb, tb)
            issue_write(l, tb, sw)
        for s in (0, 1):
            rb, tb, sg, sw = slots[s]
            wait_write(tb, sw)

    return k


def kernel(x, emb, pos_emb):
    B, L = x.shape
    V, H = emb.shape
    NW = 32
    BB = B // NW
    xi = x.astype(jnp.int32)
    xi = (xi & -512) | ((xi & 127) << 2) | ((xi >> 7) & 3)
    x_bt = xi.T.reshape(L, NW, BB).transpose(1, 0, 2)
    emb_lin = _transpose_table(emb)
    out5 = _make_kernel(B, L, V, H, NW)(x_bt, emb_lin, pos_emb)
    return out5.transpose(2, 4, 0, 1, 3).reshape(B, L, H)
